# baseline (speedup 1.0000x reference)
import jax
import jax.numpy as jnp
from jax import lax
from jax.experimental import pallas as pl
from jax.experimental.pallas import tpu as pltpu
from jax.experimental.pallas import tpu_sc as plsc

_B, _H, _D = 4096, 200, 32
_NC, _NS = 2, 16
_NW = _NC * _NS
_TH = _H // 8
_TB = _B // 128
_DG = _D // 8
_CHUNK = 128
_NCH = _TH * 8
_NBUF = 8
_LA = 5

_NBLK = 7813
_BPW = 245
_CNBUF = 8
_CLA = 5


def _conv_body(tabT_hbm, tail_hbm, out_hbm, blk, tblk, gsem, wsem):
    w = lax.axis_index("s") * _NC + lax.axis_index("c")
    base = w * _BPW
    nw = jnp.minimum(_BPW, (_NBLK - 1) - base)

    def read_start(j, s):
        pltpu.make_async_copy(
            tabT_hbm.at[:, pl.ds((base + j) * 128, 128)], blk.at[s],
            gsem.at[s]).start()

    def read_wait(s):
        pltpu.make_async_copy(
            tabT_hbm.at[:, pl.ds(0, 128)], blk.at[s], gsem.at[s]).wait()

    def transpose(s):
        rows = lax.iota(jnp.int32, 16)
        lo, hi = rows, rows + 16
        pairs = [(r, c0) for r in range(32) for c0 in range(0, 128, 16)]
        for p0 in range(0, len(pairs), 16):
            batch = pairs[p0:p0 + 16]
            srcs = [plsc.load_gather(
                        blk.at[s],
                        [lo if (c0 % 32) == 0 else hi,
                         jnp.full((16,), 4 * r + c0 // 32, jnp.int32)])
                    for r, c0 in batch]
            for (r, c0), v in zip(batch, srcs):
                tblk[s, r, pl.ds(c0, 16)] = v

    def write_start(j, s):
        pltpu.make_async_copy(
            tblk.at[s], out_hbm.at[base + j], wsem.at[s]).start()

    def write_wait(s):
        pltpu.make_async_copy(
            tblk.at[s], out_hbm.at[0], wsem.at[s]).wait()

    for j in range(_CLA):
        read_start(j, j % _CNBUF)

    @pl.loop(0, ((_BPW + _CNBUF - 1) // _CNBUF) * _CNBUF, step=_CNBUF)
    def _round(j0):
        for b in range(_CNBUF):
            j = j0 + b
            sf = (b + _CLA) % _CNBUF

            @pl.when(j + _CLA < nw)
            def _():
                @pl.when(j + _CLA >= _CNBUF)
                def _():
                    write_wait(sf)
                read_start(j + _CLA, sf)

            @pl.when(j < nw)
            def _():
                read_wait(b)
                transpose(b)
                write_start(j, b)

    for s in range(_CNBUF):
        write_wait(s)

    @pl.when(w == _NW - 1)
    def _():
        pltpu.sync_copy(tail_hbm, blk.at[0, pl.ds(0, 16)])
        pltpu.sync_copy(blk.at[0, pl.ds(0, 16)],
                        out_hbm.at[_NBLK - 1, pl.ds(0, 16)])


def _emb_body(idx_hbm, table_hbm, out_hbm, idx_v, buf, tbuf, gsem, wsem):
    w = lax.axis_index("s") * _NC + lax.axis_index("c")
    pltpu.sync_copy(idx_hbm.at[:, w], idx_v)

    def gather_start(j, s):
        th, hs = j // 8, j % 8
        for p in range(2):
            pltpu.make_async_copy(
                table_hbm.at[idx_v.at[th, hs, pl.ds(p * 64, 64)]],
                buf.at[s, pl.ds(p * 64, 64)], gsem.at[s]).start()

    def gather_start_dyn(j, s):
        th, hs = lax.div(j, 8), lax.rem(j, 8)
        for p in range(2):
            pltpu.make_async_copy(
                table_hbm.at[idx_v.at[th, hs, pl.ds(p * 64, 64)]],
                buf.at[s, pl.ds(p * 64, 64)], gsem.at[s]).start()

    def gather_wait(s):
        for p in range(2):
            pltpu.make_async_copy(
                table_hbm.at[idx_v.at[0, 0, pl.ds(p * 64, 64)]],
                buf.at[s, pl.ds(p * 64, 64)], gsem.at[s]).wait()

    def transpose(s):
        rows = lax.iota(jnp.int32, 16)
        for l in range(8):
            ridx = rows + (l * 16)
            for dblk in range(0, _D, 16):
                vs = [plsc.load_gather(
                          buf.at[s], [ridx, jnp.full((16,), d, jnp.int32)])
                      for d in range(dblk, dblk + 16)]
                for i, v in enumerate(vs):
                    d = dblk + i
                    tbuf[s, d // 8, d % 8, pl.ds(l * 16, 16)] = v

    def write_start(h, s):
        pltpu.make_async_copy(
            tbuf.at[s], out_hbm.at[h, :, w], wsem.at[s]).start()

    def write_wait(s):
        pltpu.make_async_copy(
            tbuf.at[s], out_hbm.at[0, :, w], wsem.at[s]).wait()

    for j in range(_LA):
        gather_start(j, j % _NBUF)

    @pl.loop(0, _NCH, step=_NBUF)
    def _round(j0):
        for b in range(_NBUF):
            j = j0 + b
            sf = (b + _LA) % _NBUF

            @pl.when(j + _LA < _NCH)
            def _():
                @pl.when(j + _LA >= _NBUF)
                def _():
                    write_wait(sf)
                gather_start_dyn(j + _LA, sf)

            gather_wait(b)
            transpose(b)
            write_start(j, b)

    for s in range(_NBUF):
        write_wait(s)


@jax.jit
def kernel(batch_seq, table):
    conv = pl.kernel(
        _conv_body,
        out_type=jax.ShapeDtypeStruct((_NBLK, 32, 128), jnp.float32),
        mesh=plsc.VectorSubcoreMesh(core_axis_name="c", subcore_axis_name="s"),
        scratch_types=[
            pltpu.VMEM((_CNBUF, 32, 128), jnp.float32),
            pltpu.VMEM((_CNBUF, 32, 128), jnp.float32),
            pltpu.SemaphoreType.DMA((_CNBUF,)),
            pltpu.SemaphoreType.DMA((_CNBUF,)),
        ],
        compiler_params=pltpu.CompilerParams(
            use_tc_tiling_on_sc=True, needs_layout_passes=False),
    )
    tail16 = table[(_NBLK - 1) * 128:].reshape(16, 128)
    tab_lin = conv(jnp.swapaxes(table, 0, 1),
                   tail16).reshape(_NBLK * 32 * 128 // _D, _D)
    idxn = (batch_seq.astype(jnp.int32).T
            .reshape(_TH, 8, _TB, 128).transpose(0, 2, 1, 3))
    k = pl.kernel(
        _emb_body,
        out_type=jax.ShapeDtypeStruct((_H, _DG, _TB, 8, 128), jnp.float32),
        mesh=plsc.VectorSubcoreMesh(core_axis_name="c", subcore_axis_name="s"),
        scratch_types=[
            pltpu.VMEM((_TH, 8, _CHUNK), jnp.int32),
            pltpu.VMEM((_NBUF, _CHUNK, _D), jnp.float32),
            pltpu.VMEM((_NBUF, _DG, 8, _CHUNK), jnp.float32),
            pltpu.SemaphoreType.DMA((_NBUF,)),
            pltpu.SemaphoreType.DMA((_NBUF,)),
        ],
        compiler_params=pltpu.CompilerParams(
            use_tc_tiling_on_sc=False, needs_layout_passes=False),
    )
    out5 = k(idxn, tab_lin)
    return out5.transpose(2, 4, 0, 1, 3).reshape(_B, _H, _D)


# speedup vs baseline: 2.2858x; 2.2858x over previous
import jax
import jax.numpy as jnp
from jax import lax
from jax.experimental import pallas as pl
from jax.experimental.pallas import tpu as pltpu
from jax.experimental.pallas import tpu_sc as plsc

_B, _H, _D = 4096, 200, 32
_NC, _NS = 2, 16
_NW = _NC * _NS
_TH = _H // 8
_TB = _B // 128
_DG = _D // 8
_CHUNK = 128
_NCH = _TH * 8
_NBUF = 4
_LA = 3

_NBLK = 7813
_BPW = 245
_CNBUF = 8
_CLA = 5


def _conv_body(tabT_hbm, tail_hbm, out_hbm, blk, tblk, gsem, wsem):
    w = lax.axis_index("s") * _NC + lax.axis_index("c")
    base = w * _BPW
    nw = jnp.minimum(_BPW, (_NBLK - 1) - base)

    def read_start(j, s):
        pltpu.make_async_copy(
            tabT_hbm.at[:, pl.ds((base + j) * 128, 128)],
            blk.at[s, :, pl.ds(0, 128)], gsem.at[s]).start()

    def read_wait(s):
        pltpu.make_async_copy(
            tabT_hbm.at[:, pl.ds(0, 128)], blk.at[s, :, pl.ds(0, 128)],
            gsem.at[s]).wait()

    def transpose(s):
        rows = lax.iota(jnp.int32, 16)

        @pl.loop(0, 128, step=16)
        def _t(il0):
            ilv = rows + il0
            for d0 in (0, 16):
                for k0 in range(0, 16, 8):
                    dvs = [lax.rem(rows + k, 16) + d0
                           for k in range(k0, k0 + 8)]
                    vs = [plsc.load_gather(blk.at[s], [dv, ilv])
                          for dv in dvs]
                    for dv, v in zip(dvs, vs):
                        addr = ilv * 32 + dv
                        plsc.store_scatter(
                            tblk.at[s], [addr // 128, lax.rem(addr, 128)], v)

    def write_start(j, s):
        pltpu.make_async_copy(
            tblk.at[s], out_hbm.at[base + j], wsem.at[s]).start()

    def write_wait(s):
        pltpu.make_async_copy(
            tblk.at[s], out_hbm.at[0], wsem.at[s]).wait()

    for j in range(_CLA):
        read_start(j, j % _CNBUF)

    @pl.loop(0, ((_BPW + _CNBUF - 1) // _CNBUF) * _CNBUF, step=_CNBUF)
    def _round(j0):
        for b in range(_CNBUF):
            j = j0 + b
            sf = (b + _CLA) % _CNBUF

            @pl.when(j + _CLA < nw)
            def _():
                @pl.when(j + _CLA >= _CNBUF)
                def _():
                    write_wait(sf)
                read_start(j + _CLA, sf)

            @pl.when(j < nw)
            def _():
                read_wait(b)
                transpose(b)
                write_start(j, b)

    for s in range(_CNBUF):
        write_wait(s)

    @pl.when(w == _NW - 1)
    def _():
        pltpu.sync_copy(tail_hbm, blk.at[0, pl.ds(0, 16), pl.ds(0, 128)])
        pltpu.sync_copy(blk.at[0, pl.ds(0, 16), pl.ds(0, 128)],
                        out_hbm.at[_NBLK - 1, pl.ds(0, 16)])


def _emb_body(idx_hbm, table_hbm, out_hbm, idx_v, buf, tbuf, gsem, wsem):
    w = lax.axis_index("s") * _NC + lax.axis_index("c")
    pltpu.sync_copy(idx_hbm.at[:, w], idx_v)

    def gather_start(j, s):
        th, hs = j // 8, j % 8
        for p in range(2):
            pltpu.make_async_copy(
                table_hbm.at[idx_v.at[th, hs, pl.ds(p * 64, 64)]],
                buf.at[s, pl.ds(p * 64, 64), pl.ds(0, _D)],
                gsem.at[s]).start()

    def gather_start_dyn(j, s):
        th, hs = lax.div(j, 8), lax.rem(j, 8)
        for p in range(2):
            pltpu.make_async_copy(
                table_hbm.at[idx_v.at[th, hs, pl.ds(p * 64, 64)]],
                buf.at[s, pl.ds(p * 64, 64), pl.ds(0, _D)],
                gsem.at[s]).start()

    def gather_wait(s):
        for p in range(2):
            pltpu.make_async_copy(
                table_hbm.at[idx_v.at[0, 0, pl.ds(p * 64, 64)]],
                buf.at[s, pl.ds(p * 64, 64), pl.ds(0, _D)],
                gsem.at[s]).wait()

    rows16 = lax.iota(jnp.int32, 16)

    def transpose(s):
        @pl.loop(0, _CHUNK, step=16)
        def _t(r0):
            rv = rows16 + r0
            for d0 in (0, 16):
                for k0 in range(0, 16, 8):
                    dvs = [lax.rem(rows16 + k, 16) + d0
                           for k in range(k0, k0 + 8)]
                    vs = [plsc.load_gather(buf.at[s], [rv, dv])
                          for dv in dvs]
                    for dv, v in zip(dvs, vs):
                        plsc.store_scatter(
                            tbuf.at[s], [dv // 8, lax.rem(dv, 8), rv], v)

    def write_start(h, s):
        pltpu.make_async_copy(
            tbuf.at[s], out_hbm.at[h, :, w], wsem.at[s]).start()

    def write_wait(s):
        pltpu.make_async_copy(
            tbuf.at[s], out_hbm.at[0, :, w], wsem.at[s]).wait()

    for j in range(_LA):
        gather_start(j, j % _NBUF)

    @pl.loop(0, _NCH, step=_NBUF)
    def _round(j0):
        for b in range(_NBUF):
            j = j0 + b
            sf = (b + _LA) % _NBUF

            @pl.when(j + _LA < _NCH)
            def _():
                @pl.when(j + _LA >= _NBUF)
                def _():
                    write_wait(sf)
                gather_start_dyn(j + _LA, sf)

            gather_wait(b)
            transpose(b)
            write_start(j, b)

    for s in range(_NBUF):
        write_wait(s)


@jax.jit
def kernel(batch_seq, table):
    conv = pl.kernel(
        _conv_body,
        out_type=jax.ShapeDtypeStruct((_NBLK, 32, 128), jnp.float32),
        mesh=plsc.VectorSubcoreMesh(core_axis_name="c", subcore_axis_name="s"),
        scratch_types=[
            pltpu.VMEM((_CNBUF, 32, 128), jnp.float32),
            pltpu.VMEM((_CNBUF, 32, 128), jnp.float32),
            pltpu.SemaphoreType.DMA((_CNBUF,)),
            pltpu.SemaphoreType.DMA((_CNBUF,)),
        ],
        compiler_params=pltpu.CompilerParams(
            use_tc_tiling_on_sc=True, needs_layout_passes=False),
    )
    tail16 = table[(_NBLK - 1) * 128:].reshape(16, 128)
    tab_lin = conv(jnp.swapaxes(table, 0, 1),
                   tail16).reshape(_NBLK * 32 * 128 // _D, _D)
    idxn = (batch_seq.astype(jnp.int32).T
            .reshape(_TH, 8, _TB, 128).transpose(0, 2, 1, 3))
    k = pl.kernel(
        _emb_body,
        out_type=jax.ShapeDtypeStruct((_H, _DG, _TB, 8, 128), jnp.float32),
        mesh=plsc.VectorSubcoreMesh(core_axis_name="c", subcore_axis_name="s"),
        scratch_types=[
            pltpu.VMEM((_TH, 8, _CHUNK), jnp.int32),
            pltpu.VMEM((_NBUF, _CHUNK, _D), jnp.float32),
            pltpu.VMEM((_NBUF, _DG, 8, _CHUNK), jnp.float32),
            pltpu.SemaphoreType.DMA((_NBUF,)),
            pltpu.SemaphoreType.DMA((_NBUF,)),
        ],
        compiler_params=pltpu.CompilerParams(
            use_tc_tiling_on_sc=False, needs_layout_passes=False),
    )
    out5 = k(idxn, tab_lin)
    return out5.transpose(2, 4, 0, 1, 3).reshape(_B, _H, _D)


# speedup vs baseline: 2.4802x; 1.0850x over previous
import jax
import jax.numpy as jnp
from jax import lax
from jax.experimental import pallas as pl
from jax.experimental.pallas import tpu as pltpu
from jax.experimental.pallas import tpu_sc as plsc

_B, _H, _D = 4096, 200, 32
_NC, _NS = 2, 16
_NW = _NC * _NS
_TH = _H // 8
_TB = _B // 128
_DG = _D // 8
_CHUNK = 128
_NCH = _TH * 8
_NBUF = 4
_LA = 3

_NBLK = 7813
_BPW = 245
_CNBUF = 8
_CLA = 5


def _conv_body(tabT_hbm, tail_hbm, out_hbm, blk, tblk, gsem, wsem):
    w = lax.axis_index("s") * _NC + lax.axis_index("c")
    base = w * _BPW
    nw = jnp.minimum(_BPW, (_NBLK - 1) - base)

    def read_start(j, s):
        pltpu.make_async_copy(
            tabT_hbm.at[:, pl.ds((base + j) * 128, 128)],
            blk.at[s, :, pl.ds(0, 128)], gsem.at[s]).start()

    def read_wait(s):
        pltpu.make_async_copy(
            tabT_hbm.at[:, pl.ds(0, 128)], blk.at[s, :, pl.ds(0, 128)],
            gsem.at[s]).wait()

    def transpose(s):
        rows = lax.iota(jnp.int32, 16)

        @pl.loop(0, 128, step=16)
        def _t(il0):
            ilv = rows + il0
            il32 = ilv * 32
            dloc = [lax.rem(rows + k, 16) for k in range(16)]
            for d0 in (0, 16):
                for k0 in range(0, 16, 8):
                    dvs = [dloc[k] + d0 for k in range(k0, k0 + 8)]
                    vs = [plsc.load_gather(blk.at[s], [dv, ilv])
                          for dv in dvs]
                    for dv, v in zip(dvs, vs):
                        addr = il32 + dv
                        plsc.store_scatter(
                            tblk.at[s], [addr // 128, lax.rem(addr, 128)], v)

    def write_start(j, s):
        pltpu.make_async_copy(
            tblk.at[s], out_hbm.at[base + j], wsem.at[s]).start()

    def write_wait(s):
        pltpu.make_async_copy(
            tblk.at[s], out_hbm.at[0], wsem.at[s]).wait()

    for j in range(_CLA):
        read_start(j, j % _CNBUF)

    @pl.loop(0, ((_BPW + _CNBUF - 1) // _CNBUF) * _CNBUF, step=_CNBUF)
    def _round(j0):
        for b in range(_CNBUF):
            j = j0 + b
            sf = (b + _CLA) % _CNBUF

            @pl.when(j + _CLA < nw)
            def _():
                @pl.when(j + _CLA >= _CNBUF)
                def _():
                    write_wait(sf)
                read_start(j + _CLA, sf)

            @pl.when(j < nw)
            def _():
                read_wait(b)
                transpose(b)
                write_start(j, b)

    for s in range(_CNBUF):
        write_wait(s)

    @pl.when(w == _NW - 1)
    def _():
        pltpu.sync_copy(tail_hbm, blk.at[0, pl.ds(0, 16), pl.ds(0, 128)])
        pltpu.sync_copy(blk.at[0, pl.ds(0, 16), pl.ds(0, 128)],
                        out_hbm.at[_NBLK - 1, pl.ds(0, 16)])


def _emb_body(idx_hbm, table_hbm, out_hbm, idx_v, buf, tbuf, gsem, wsem):
    w = lax.axis_index("s") * _NC + lax.axis_index("c")
    pltpu.sync_copy(idx_hbm.at[:, w], idx_v)

    def gather_start(j, s):
        th, hs = j // 8, j % 8
        for p in range(2):
            pltpu.make_async_copy(
                table_hbm.at[idx_v.at[th, hs, pl.ds(p * 64, 64)]],
                buf.at[s, pl.ds(p * 64, 64), pl.ds(0, _D)],
                gsem.at[s]).start()

    def gather_start_dyn(j, s):
        th, hs = lax.div(j, 8), lax.rem(j, 8)
        for p in range(2):
            pltpu.make_async_copy(
                table_hbm.at[idx_v.at[th, hs, pl.ds(p * 64, 64)]],
                buf.at[s, pl.ds(p * 64, 64), pl.ds(0, _D)],
                gsem.at[s]).start()

    def gather_wait(s):
        for p in range(2):
            pltpu.make_async_copy(
                table_hbm.at[idx_v.at[0, 0, pl.ds(p * 64, 64)]],
                buf.at[s, pl.ds(p * 64, 64), pl.ds(0, _D)],
                gsem.at[s]).wait()

    rows16 = lax.iota(jnp.int32, 16)

    def transpose(s):
        @pl.loop(0, _CHUNK, step=16)
        def _t(r0):
            rv = rows16 + r0
            for d0 in (0, 16):
                for k0 in range(0, 16, 8):
                    dvs = [lax.rem(rows16 + k, 16) + d0
                           for k in range(k0, k0 + 8)]
                    vs = [plsc.load_gather(buf.at[s], [rv, dv])
                          for dv in dvs]
                    for dv, v in zip(dvs, vs):
                        plsc.store_scatter(
                            tbuf.at[s], [dv // 8, lax.rem(dv, 8), rv], v)

    def write_start(h, s):
        pltpu.make_async_copy(
            tbuf.at[s], out_hbm.at[h, :, w], wsem.at[s]).start()

    def write_wait(s):
        pltpu.make_async_copy(
            tbuf.at[s], out_hbm.at[0, :, w], wsem.at[s]).wait()

    for j in range(_LA):
        gather_start(j, j % _NBUF)

    @pl.loop(0, _NCH, step=_NBUF)
    def _round(j0):
        for b in range(_NBUF):
            j = j0 + b
            sf = (b + _LA) % _NBUF

            @pl.when(j + _LA < _NCH)
            def _():
                @pl.when(j + _LA >= _NBUF)
                def _():
                    write_wait(sf)
                gather_start_dyn(j + _LA, sf)

            gather_wait(b)
            transpose(b)
            write_start(j, b)

    for s in range(_NBUF):
        write_wait(s)


@jax.jit
def kernel(batch_seq, table):
    conv = pl.kernel(
        _conv_body,
        out_type=jax.ShapeDtypeStruct((_NBLK, 32, 128), jnp.float32),
        mesh=plsc.VectorSubcoreMesh(core_axis_name="c", subcore_axis_name="s"),
        scratch_types=[
            pltpu.VMEM((_CNBUF, 32, 128), jnp.float32),
            pltpu.VMEM((_CNBUF, 32, 128), jnp.float32),
            pltpu.SemaphoreType.DMA((_CNBUF,)),
            pltpu.SemaphoreType.DMA((_CNBUF,)),
        ],
        compiler_params=pltpu.CompilerParams(
            use_tc_tiling_on_sc=True, needs_layout_passes=False),
    )
    tail16 = table[(_NBLK - 1) * 128:].reshape(16, 128)
    tab_lin = conv(jnp.swapaxes(table, 0, 1),
                   tail16).reshape(_NBLK * 32 * 128 // _D, _D)
    idxn = (batch_seq.astype(jnp.int32).T
            .reshape(_TH, 8, _TB, 128).transpose(0, 2, 1, 3))
    k = pl.kernel(
        _emb_body,
        out_type=jax.ShapeDtypeStruct((_H, _DG, _TB, 8, 128), jnp.float32),
        mesh=plsc.VectorSubcoreMesh(core_axis_name="c", subcore_axis_name="s"),
        scratch_types=[
            pltpu.VMEM((_TH, 8, _CHUNK), jnp.int32),
            pltpu.VMEM((_NBUF, _CHUNK, _D), jnp.float32),
            pltpu.VMEM((_NBUF, _DG, 8, _CHUNK), jnp.float32),
            pltpu.SemaphoreType.DMA((_NBUF,)),
            pltpu.SemaphoreType.DMA((_NBUF,)),
        ],
        compiler_params=pltpu.CompilerParams(
            use_tc_tiling_on_sc=False, needs_layout_passes=False),
    )
    out5 = k(idxn, tab_lin)
    return out5.transpose(2, 4, 0, 1, 3).reshape(_B, _H, _D)


# speedup vs baseline: 2.4873x; 1.0029x over previous
import jax
import jax.numpy as jnp
from jax import lax
from jax.experimental import pallas as pl
from jax.experimental.pallas import tpu as pltpu
from jax.experimental.pallas import tpu_sc as plsc

_B, _H, _D = 4096, 200, 32
_NC, _NS = 2, 16
_NW = _NC * _NS
_TH = _H // 8
_TB = _B // 128
_DG = _D // 8
_CHUNK = 128
_NCH = _TH * 8
_NBUF = 4
_LA = 3

_NBLK = 7813
_BPW = 245
_CNBUF = 8
_CLA = 5


def _conv_body(tabT_hbm, tail_hbm, out_hbm, blk, tblk, gsem, wsem):
    w = lax.axis_index("s") * _NC + lax.axis_index("c")
    base = w * _BPW
    nw = jnp.minimum(_BPW, (_NBLK - 1) - base)

    def read_start(j, s):
        pltpu.make_async_copy(
            tabT_hbm.at[:, pl.ds((base + j) * 128, 128)],
            blk.at[s, :, pl.ds(0, 128)], gsem.at[s]).start()

    def read_wait(s):
        pltpu.make_async_copy(
            tabT_hbm.at[:, pl.ds(0, 128)], blk.at[s, :, pl.ds(0, 128)],
            gsem.at[s]).wait()

    def transpose(s):
        rows = lax.iota(jnp.int32, 16)

        @pl.loop(0, 128, step=16)
        def _t(il0):
            ilv = rows + il0
            il32 = ilv * 32
            dloc = [lax.rem(rows + k, 16) for k in range(16)]
            for d0 in (0, 16):
                for k0 in range(0, 16, 8):
                    dvs = [dloc[k] + d0 for k in range(k0, k0 + 8)]
                    vs = [plsc.load_gather(blk.at[s], [dv, ilv])
                          for dv in dvs]
                    for dv, v in zip(dvs, vs):
                        addr = il32 + dv
                        plsc.store_scatter(
                            tblk.at[s], [addr // 128, lax.rem(addr, 128)], v)

    def write_start(j, s):
        pltpu.make_async_copy(
            tblk.at[s], out_hbm.at[base + j], wsem.at[s]).start()

    def write_wait(s):
        pltpu.make_async_copy(
            tblk.at[s], out_hbm.at[0], wsem.at[s]).wait()

    for j in range(_CLA):
        read_start(j, j % _CNBUF)

    @pl.loop(0, ((_BPW + _CNBUF - 1) // _CNBUF) * _CNBUF, step=_CNBUF)
    def _round(j0):
        for b in range(_CNBUF):
            j = j0 + b
            sf = (b + _CLA) % _CNBUF

            @pl.when(j + _CLA < nw)
            def _():
                @pl.when(j + _CLA >= _CNBUF)
                def _():
                    write_wait(sf)
                read_start(j + _CLA, sf)

            @pl.when(j < nw)
            def _():
                read_wait(b)
                transpose(b)
                write_start(j, b)

    for s in range(_CNBUF):
        write_wait(s)

    @pl.when(w == _NW - 1)
    def _():
        pltpu.sync_copy(tail_hbm, blk.at[0, pl.ds(0, 16), pl.ds(0, 128)])
        pltpu.sync_copy(blk.at[0, pl.ds(0, 16), pl.ds(0, 128)],
                        out_hbm.at[_NBLK - 1, pl.ds(0, 16)])


def _emb_body(idx_hbm, table_hbm, out_hbm, idx_v, buf, tbuf, gsem, wsem):
    w = lax.axis_index("s") * _NC + lax.axis_index("c")
    pltpu.sync_copy(idx_hbm.at[:, w], idx_v)

    def gather_start(j, s):
        th, hs = j // 8, j % 8
        for p in range(2):
            pltpu.make_async_copy(
                table_hbm.at[idx_v.at[th, hs, pl.ds(p * 64, 64)]],
                buf.at[s, pl.ds(p * 64, 64), pl.ds(0, _D)],
                gsem.at[s]).start()

    def gather_start_dyn(j, s):
        th, hs = lax.div(j, 8), lax.rem(j, 8)
        for p in range(2):
            pltpu.make_async_copy(
                table_hbm.at[idx_v.at[th, hs, pl.ds(p * 64, 64)]],
                buf.at[s, pl.ds(p * 64, 64), pl.ds(0, _D)],
                gsem.at[s]).start()

    def gather_wait(s):
        for p in range(2):
            pltpu.make_async_copy(
                table_hbm.at[idx_v.at[0, 0, pl.ds(p * 64, 64)]],
                buf.at[s, pl.ds(p * 64, 64), pl.ds(0, _D)],
                gsem.at[s]).wait()

    rows16 = lax.iota(jnp.int32, 16)

    def transpose(s):
        @pl.loop(0, _CHUNK, step=16)
        def _t(r0):
            rv = rows16 + r0
            dloc = [lax.rem(rows16 + k, 16) for k in range(16)]
            dls = [lax.rem(dl, 8) for dl in dloc]
            for d0 in (0, 16):
                for k0 in range(0, 16, 8):
                    dvs = [dloc[k] + d0 for k in range(k0, k0 + 8)]
                    vs = [plsc.load_gather(buf.at[s], [rv, dv])
                          for dv in dvs]
                    for (k, dv), v in zip(
                            [(k, dv) for k, dv in
                             zip(range(k0, k0 + 8), dvs)], vs):
                        plsc.store_scatter(
                            tbuf.at[s], [dv // 8, dls[k], rv], v)

    def write_start(h, s):
        pltpu.make_async_copy(
            tbuf.at[s], out_hbm.at[h, :, w], wsem.at[s]).start()

    def write_wait(s):
        pltpu.make_async_copy(
            tbuf.at[s], out_hbm.at[0, :, w], wsem.at[s]).wait()

    for j in range(_LA):
        gather_start(j, j % _NBUF)

    @pl.loop(0, _NCH, step=_NBUF)
    def _round(j0):
        for b in range(_NBUF):
            j = j0 + b
            sf = (b + _LA) % _NBUF

            @pl.when(j + _LA < _NCH)
            def _():
                @pl.when(j + _LA >= _NBUF)
                def _():
                    write_wait(sf)
                gather_start_dyn(j + _LA, sf)

            gather_wait(b)
            transpose(b)
            write_start(j, b)

    for s in range(_NBUF):
        write_wait(s)


@jax.jit
def kernel(batch_seq, table):
    conv = pl.kernel(
        _conv_body,
        out_type=jax.ShapeDtypeStruct((_NBLK, 32, 128), jnp.float32),
        mesh=plsc.VectorSubcoreMesh(core_axis_name="c", subcore_axis_name="s"),
        scratch_types=[
            pltpu.VMEM((_CNBUF, 32, 128), jnp.float32),
            pltpu.VMEM((_CNBUF, 32, 128), jnp.float32),
            pltpu.SemaphoreType.DMA((_CNBUF,)),
            pltpu.SemaphoreType.DMA((_CNBUF,)),
        ],
        compiler_params=pltpu.CompilerParams(
            use_tc_tiling_on_sc=True, needs_layout_passes=False),
    )
    tail16 = table[(_NBLK - 1) * 128:].reshape(16, 128)
    tab_lin = conv(jnp.swapaxes(table, 0, 1),
                   tail16).reshape(_NBLK * 32 * 128 // _D, _D)
    idxn = (batch_seq.astype(jnp.int32).T
            .reshape(_TH, 8, _TB, 128).transpose(0, 2, 1, 3))
    k = pl.kernel(
        _emb_body,
        out_type=jax.ShapeDtypeStruct((_H, _DG, _TB, 8, 128), jnp.float32),
        mesh=plsc.VectorSubcoreMesh(core_axis_name="c", subcore_axis_name="s"),
        scratch_types=[
            pltpu.VMEM((_TH, 8, _CHUNK), jnp.int32),
            pltpu.VMEM((_NBUF, _CHUNK, _D), jnp.float32),
            pltpu.VMEM((_NBUF, _DG, 8, _CHUNK), jnp.float32),
            pltpu.SemaphoreType.DMA((_NBUF,)),
            pltpu.SemaphoreType.DMA((_NBUF,)),
        ],
        compiler_params=pltpu.CompilerParams(
            use_tc_tiling_on_sc=False, needs_layout_passes=False),
    )
    out5 = k(idxn, tab_lin)
    return out5.transpose(2, 4, 0, 1, 3).reshape(_B, _H, _D)
